# baseline (speedup 1.0000x reference)
import functools

import jax
import jax.numpy as jnp
from jax import lax
from jax.experimental import pallas as pl
from jax.experimental.pallas import tpu as pltpu
from jax.experimental.pallas import tpu_sc as plsc

_IW = 128
_RPU = 4
_CB = _RPU * _IW
_TC = 896


def _worker_id(nc):
  return lax.axis_index("s") * nc + lax.axis_index("c")


@functools.cache
def _make_pack(v: int, d: int):
  info = plsc.get_sparse_core_info()
  nc, ns, nl = info.num_cores, info.num_subcores, info.num_lanes
  nw = nc * ns
  nch = v // _TC
  tail = v - nch * _TC
  pairs = (nch + 2 * nw - 1) // (2 * nw)
  rpc = _TC * d // 128

  mesh = plsc.VectorSubcoreMesh(core_axis_name="c", subcore_axis_name="s")

  @functools.partial(
      pl.kernel,
      mesh=mesh,
      out_type=jax.ShapeDtypeStruct((v * d // 128, 128), jnp.float32),
      compiler_params=pltpu.CompilerParams(needs_layout_passes=False),
      scratch_types=[
          pltpu.VMEM((d, _TC), jnp.float32),
          pltpu.VMEM((d, _TC), jnp.float32),
          pltpu.VMEM((rpc, 128), jnp.float32),
          pltpu.VMEM((rpc, 128), jnp.float32),
          pltpu.SemaphoreType.DMA,
          pltpu.SemaphoreType.DMA,
          pltpu.SemaphoreType.DMA,
          pltpu.SemaphoreType.DMA,
      ],
  )
  def pack(tt_hbm, tail_hbm, out_hbm, buf0, buf1, ov0, ov1,
           ss0, ss1, sw0, sw1):
    wid = _worker_id(nc)
    lane = lax.iota(jnp.int32, nl)
    cols = [(lane + dd) & (d - 1) for dd in range(d)]
    bufs, ovs = (buf0, buf1), (ov0, ov1)
    ssems, wsems = (ss0, ss1), (sw0, sw1)

    def fire_stage(c, p):
      pltpu.async_copy(tt_hbm.at[:, pl.ds(c * _TC, _TC)], bufs[p], ssems[p])

    def wait_stage(p):
      pltpu.make_async_copy(
          tt_hbm.at[:, pl.ds(0, _TC)], bufs[p], ssems[p]).wait()

    def fire_wb(c, p):
      pltpu.async_copy(ovs[p], out_hbm.at[pl.ds(c * rpc, rpc), :], wsems[p])

    def wait_wb(p):
      pltpu.make_async_copy(
          ovs[p], out_hbm.at[pl.ds(0, rpc), :], wsems[p]).wait()

    def transpose(p):
      buf, outv = bufs[p], ovs[p]

      def _tj(j, carry):
        rows = j * nl + lane
        rdiv = rows >> 2
        rmod = (rows & 3) << 5
        vals = [plsc.load_gather(buf, [cols[dd], rows]) for dd in range(d)]
        for dd in range(d):
          plsc.store_scatter(outv, [rdiv, rmod + cols[dd]], vals[dd])
        return carry

      lax.fori_loop(0, _TC // nl, _tj, 0)

    def half(c_this, c_next2, p, first):
      @pl.when(c_this < nch)
      def _():
        wait_stage(p)

        @pl.when(jnp.logical_not(first))
        def _():
          wait_wb(p)

        transpose(p)
        fire_wb(c_this, p)

        @pl.when(c_next2 < nch)
        def _():
          fire_stage(c_next2, p)

    fire_stage(wid, 0)

    @pl.when(wid + nw < nch)
    def _():
      fire_stage(wid + nw, 1)

    def body(i, carry):
      ca = wid + (2 * i) * nw
      cb = wid + (2 * i + 1) * nw
      half(ca, wid + (2 * i + 2) * nw, 0, i == 0)
      half(cb, wid + (2 * i + 3) * nw, 1, i == 0)
      return carry

    lax.fori_loop(0, pairs, body, 0)
    wait_wb(0)
    wait_wb(1)

    if tail:
      trows = tail * d // 128

      @pl.when(wid == 0)
      def _():
        pltpu.sync_copy(tail_hbm, ov0.at[pl.ds(0, trows), :])
        pltpu.sync_copy(ov0.at[pl.ds(0, trows), :],
                        out_hbm.at[pl.ds(nch * rpc, trows), :])

  return pack


@functools.cache
def _make_gather(b: int, f: int, v: int, d: int):
  info = plsc.get_sparse_core_info()
  nc, ns, nl = info.num_cores, info.num_subcores, info.num_lanes
  nw = nc * ns
  assert b == nw * _CB and d % nl == 0 and nl == 16

  mesh = plsc.VectorSubcoreMesh(core_axis_name="c", subcore_axis_name="s")

  @functools.partial(
      pl.kernel,
      mesh=mesh,
      out_type=jax.ShapeDtypeStruct((f, d // 8, b // 128, 8, 128),
                                    jnp.float32),
      compiler_params=pltpu.CompilerParams(
          use_tc_tiling_on_sc=False, needs_layout_passes=False),
      scratch_types=[
          pltpu.VMEM((f, _RPU, _IW), jnp.int32),
          pltpu.VMEM((_CB, d), jnp.float32),
          pltpu.VMEM((_CB, d), jnp.float32),
          pltpu.VMEM((d // 8, _CB // 128, 8, 128), jnp.float32),
          pltpu.VMEM((d // 8, _CB // 128, 8, 128), jnp.float32),
          pltpu.SemaphoreType.DMA,
          pltpu.SemaphoreType.DMA,
          pltpu.SemaphoreType.DMA,
          pltpu.SemaphoreType.DMA,
      ],
  )
  def gather(idx_hbm, table_hbm, out_hbm, idx_all, buf0, buf1, ov0, ov1,
             sg0, sg1, sw0, sw1):
    wid = _worker_id(nc)
    b0 = wid * _CB
    pltpu.sync_copy(idx_hbm.at[:, pl.ds(wid * _RPU, _RPU), :], idx_all)

    bufs, ovs = (buf0, buf1), (ov0, ov1)
    gsems, wsems = (sg0, sg1), (sw0, sw1)
    lane = lax.iota(jnp.int32, nl)
    cols = [(lane + dd) & (d - 1) for dd in range(d)]

    def fire(u, p):
      for r in range(_RPU):
        pltpu.async_copy(
            table_hbm.at[idx_all.at[u, r]],
            bufs[p].at[pl.ds(r * _IW, _IW), :],
            gsems[p],
        )

    def wait(p):
      for r in range(_RPU):
        pltpu.make_async_copy(
            table_hbm.at[idx_all.at[0, r]],
            bufs[p].at[pl.ds(r * _IW, _IW), :],
            gsems[p],
        ).wait()

    tb0 = wid * (_CB // 128)
    tdv = [c >> 3 for c in cols]
    d8v = [c & 7 for c in cols]

    def fire_wb(u, p):
      pltpu.async_copy(
          ovs[p], out_hbm.at[u, :, pl.ds(tb0, _CB // 128), :, :], wsems[p])

    def wait_wb(p):
      pltpu.make_async_copy(
          ovs[p], out_hbm.at[0, :, pl.ds(0, _CB // 128), :, :],
          wsems[p]).wait()

    def transpose(p):
      buf, outv = bufs[p], ovs[p]

      def _tj(j, carry):
        rows = j * nl + lane
        tb = rows >> 7
        b128 = rows & 127
        vals = [plsc.load_gather(buf, [rows, cols[dd]]) for dd in range(d)]
        for dd in range(d):
          plsc.store_scatter(outv, [tdv[dd], tb, d8v[dd], b128], vals[dd])
        return carry

      lax.fori_loop(0, _CB // nl, _tj, 0)

    def unit(u, p, prefetch, first):
      wait(p)

      @pl.when(prefetch)
      def _():
        fire(u + 1, 1 - p)

      @pl.when(jnp.logical_not(first))
      def _():
        wait_wb(p)

      transpose(p)
      fire_wb(u, p)

    npairs = f // 2
    fire(0, 0)

    def body(j, carry):
      unit(2 * j, 0, jnp.bool_(True), j == 0)
      unit(2 * j + 1, 1, j < npairs - 1, j == 0)
      return carry

    lax.fori_loop(0, npairs, body, 0)
    wait_wb(0)
    wait_wb(1)

  return gather


def kernel(ids, table):
  b, f = ids.shape
  v, d = table.shape
  ids3 = ids.T.reshape(f, b // _IW, _IW)
  nfull = (v // _TC) * _TC
  tail_rm = table[nfull:, :].reshape((v - nfull) * d // 128, 128)
  packed = _make_pack(v, d)(table.T, tail_rm)
  table_rm = packed.reshape(v, d)
  out5 = _make_gather(b, f, v, d)(ids3, table_rm)
  return out5.transpose(2, 4, 0, 1, 3).reshape(b, f, d)
